# Initial kernel scaffold; baseline (speedup 1.0000x reference)
import jax
import jax.numpy as jnp
from jax.experimental import pallas as pl


def kernel(x, Wg, bg, W1, b1, ln_g, ln_b, W2, b2, out_g, out_b):
    raise NotImplementedError("write your pallas kernel here")


# speedup vs baseline: 14.7340x; 14.7340x over previous
import functools

import jax
import jax.numpy as jnp
from jax.experimental import pallas as pl
from jax.experimental.pallas import tpu as pltpu

T, D, H, O, E, K = 1024, 1024, 512, 1024, 64, 2
NSLOT = K * T
RB = 128
CH = T // RB
NROWS = NSLOT + E * 8 + RB


def _route_kernel(x_ref, wg_ref, bg_ref,
                  pos0_ref, pos1_ref, w0_ref, w1_ref,
                  start_ref, cnt_ref, aux_ref):
    x = x_ref[...]
    logits = jnp.dot(x, wg_ref[...], preferred_element_type=jnp.float32)
    logits = logits + bg_ref[...]
    m = jnp.max(logits, axis=-1, keepdims=True)
    p = jnp.exp(logits - m)
    probs = p / jnp.sum(p, axis=-1, keepdims=True)

    lane = jax.lax.broadcasted_iota(jnp.int32, (T, E), 1)
    m1 = jnp.max(probs, axis=-1, keepdims=True)
    i1 = jnp.min(jnp.where(probs == m1, lane, E), axis=-1, keepdims=True)
    probs2 = jnp.where(lane == i1, -1.0, probs)
    m2 = jnp.max(probs2, axis=-1, keepdims=True)
    i2 = jnp.min(jnp.where(probs2 == m2, lane, E), axis=-1, keepdims=True)
    s = m1 + m2
    w0_ref[...] = m1 / s
    w1_ref[...] = m2 / s

    e_flat = jnp.concatenate([i1, i2], axis=0)
    lane2 = jax.lax.broadcasted_iota(jnp.int32, (NSLOT, E), 1)
    onehot = (lane2 == e_flat).astype(jnp.float32)
    counts = jnp.sum(onehot, axis=0, keepdims=True)

    counts_i = counts.astype(jnp.int32)
    pcnt = ((counts_i + 7) // 8) * 8
    r64 = jax.lax.broadcasted_iota(jnp.int32, (E, E), 0)
    c64 = jax.lax.broadcasted_iota(jnp.int32, (E, E), 1)
    lower64 = (r64 < c64).astype(jnp.float32)
    starts = jnp.dot(pcnt.astype(jnp.float32), lower64,
                     precision=jax.lax.Precision.HIGHEST)

    r2 = jax.lax.broadcasted_iota(jnp.int32, (NSLOT, NSLOT), 0)
    c2 = jax.lax.broadcasted_iota(jnp.int32, (NSLOT, NSLOT), 1)
    tril = (r2 > c2).astype(jnp.float32)
    before = jnp.dot(tril, onehot, precision=jax.lax.Precision.HIGHEST)
    rank = jnp.sum(before * onehot, axis=-1, keepdims=True)

    start_per_slot = jnp.dot(onehot, starts.T,
                             precision=jax.lax.Precision.HIGHEST)
    pos_flat = (start_per_slot + rank).astype(jnp.int32)
    pos0_ref[...] = pos_flat[:T]
    pos1_ref[...] = pos_flat[T:]
    start_ref[...] = starts.astype(jnp.int32)
    cnt_ref[...] = counts_i

    imp = jnp.sum(probs, axis=0, keepdims=True)
    imp_mu = jnp.mean(imp)
    imp_var = jnp.sum((imp - imp_mu) ** 2) / (E - 1)
    imp_loss = imp_var / (imp_mu + 1e-6) ** 2
    load = counts / T
    load_mu = jnp.mean(load)
    load_var = jnp.sum((load - load_mu) ** 2) / (E - 1)
    load_loss = load_var / (load_mu + 1e-6) ** 2
    aux_ref[...] = jnp.full((1, 1), imp_loss + load_loss, dtype=jnp.float32)


def _dispatch_kernel(pos0_ref, pos1_ref, x_ref, xs_ref):
    pi = jax.lax.broadcasted_iota(jnp.int32, (NROWS, T), 0)
    p0 = pos0_ref[...].reshape(1, T)
    p1 = pos1_ref[...].reshape(1, T)
    sel = ((pi == p0) | (pi == p1)).astype(jnp.float32)
    xs_ref[...] = jnp.dot(sel, x_ref[...], preferred_element_type=jnp.float32)


def _ffn_kernel(start_ref, cnt_ref, xs_ref, w1_ref, b1_ref, g_ref, bb_ref,
                w2_ref, b2_ref, y_ref):
    e = pl.program_id(0)
    j = pl.program_id(1)
    st = pl.multiple_of(start_ref[e], 8)
    cn = cnt_ref[e]

    @pl.when(j * RB < cn)
    def _():
        base = st + j * RB
        xs = xs_ref[pl.ds(base, RB), :]
        h = jnp.dot(xs, w1_ref[0], preferred_element_type=jnp.float32)
        h = h + b1_ref[0]
        mu = jnp.mean(h, axis=-1, keepdims=True)
        var = jnp.mean((h - mu) ** 2, axis=-1, keepdims=True)
        h = (h - mu) * jax.lax.rsqrt(var + 1e-5) * g_ref[0] + bb_ref[0]
        h = 0.5 * h * (1.0 + jax.lax.erf(h * (2.0 ** -0.5)))
        y = jnp.dot(h, w2_ref[0], preferred_element_type=jnp.float32)
        y_ref[pl.ds(base, RB), :] = y + b2_ref[0]


def _combine_kernel(pos0_ref, pos1_ref, w0_ref, w1_ref, y_ref,
                    og_ref, ob_ref, out_ref):
    pi = jax.lax.broadcasted_iota(jnp.int32, (T, NROWS), 1)
    c0 = jnp.where(pi == pos0_ref[...], w0_ref[...], 0.0)
    c1 = jnp.where(pi == pos1_ref[...], w1_ref[...], 0.0)
    yv = y_ref[...]
    yv = jnp.where(jnp.isfinite(yv), yv, 0.0)
    comb = jnp.dot(c0 + c1, yv, preferred_element_type=jnp.float32)
    mu = jnp.mean(comb, axis=-1, keepdims=True)
    var = jnp.mean((comb - mu) ** 2, axis=-1, keepdims=True)
    out_ref[...] = ((comb - mu) * jax.lax.rsqrt(var + 1e-5) * og_ref[...]
                    + ob_ref[...])


def kernel(x, Wg, bg, W1, b1, ln_g, ln_b, W2, b2, out_g, out_b):
    f32, i32 = jnp.float32, jnp.int32
    pos0, pos1, w0, w1, start, cnt, aux = pl.pallas_call(
        _route_kernel,
        out_shape=(
            jax.ShapeDtypeStruct((T, 1), i32),
            jax.ShapeDtypeStruct((T, 1), i32),
            jax.ShapeDtypeStruct((T, 1), f32),
            jax.ShapeDtypeStruct((T, 1), f32),
            jax.ShapeDtypeStruct((1, E), i32),
            jax.ShapeDtypeStruct((1, E), i32),
            jax.ShapeDtypeStruct((1, 1), f32),
        ),
    )(x, Wg, bg.reshape(1, E))

    xs = pl.pallas_call(
        _dispatch_kernel,
        out_shape=jax.ShapeDtypeStruct((NROWS, D), f32),
    )(pos0, pos1, x)

    grid_spec = pltpu.PrefetchScalarGridSpec(
        num_scalar_prefetch=2,
        grid=(E, CH),
        in_specs=[
            pl.BlockSpec((NROWS, D), lambda e, j, s, c: (0, 0)),
            pl.BlockSpec((1, D, H), lambda e, j, s, c: (e, 0, 0)),
            pl.BlockSpec((1, 1, H), lambda e, j, s, c: (e, 0, 0)),
            pl.BlockSpec((1, 1, H), lambda e, j, s, c: (e, 0, 0)),
            pl.BlockSpec((1, 1, H), lambda e, j, s, c: (e, 0, 0)),
            pl.BlockSpec((1, H, O), lambda e, j, s, c: (e, 0, 0)),
            pl.BlockSpec((1, 1, O), lambda e, j, s, c: (e, 0, 0)),
        ],
        out_specs=pl.BlockSpec((NROWS, O), lambda e, j, s, c: (0, 0)),
    )
    y = pl.pallas_call(
        _ffn_kernel,
        grid_spec=grid_spec,
        out_shape=jax.ShapeDtypeStruct((NROWS, O), f32),
    )(start.reshape(E), cnt.reshape(E), xs, W1,
      b1.reshape(E, 1, H), ln_g.reshape(E, 1, H), ln_b.reshape(E, 1, H),
      W2, b2.reshape(E, 1, O))

    out = pl.pallas_call(
        _combine_kernel,
        out_shape=jax.ShapeDtypeStruct((T, O), f32),
    )(pos0, pos1, w0, w1, y, out_g.reshape(1, O), out_b.reshape(1, O))

    return out, aux.reshape(())
